# Initial kernel scaffold; baseline (speedup 1.0000x reference)
import jax
import jax.numpy as jnp
from jax.experimental import pallas as pl


def kernel(x, w1, g1, b1, w2, g2, b2):
    raise NotImplementedError("write your pallas kernel here")


# speedup vs baseline: 4.1564x; 4.1564x over previous
import functools

import jax
import jax.numpy as jnp
from jax.experimental import pallas as pl
from jax.experimental.pallas import tpu as pltpu

EPS = 1e-5
IMG_BLOCK = 4
TILE_M = 4096
VMEM_LIMIT = 100 * 1024 * 1024


def _conv3x3_body(x_ref, w_ref, scale_ref, shift_ref, y_ref, stats_ref, *,
                  pre_bn_relu):
    B, H, W, C = x_ref.shape
    Cp = y_ref.shape[-1]

    xv = x_ref[...].astype(jnp.float32)
    if pre_bn_relu:
        xv = jnp.maximum(xv * scale_ref[...] + shift_ref[...], 0.0)
    xv = xv.astype(jnp.bfloat16)

    zc = jnp.zeros((B, H, 1, C), jnp.bfloat16)
    xp = jnp.concatenate([zc, xv, zc], axis=2)
    zr = jnp.zeros((B, 1, W + 2, C), jnp.bfloat16)
    xp = jnp.concatenate([zr, xp, zr], axis=1)

    cols = [xp[:, kh:kh + H, kw:kw + W, :].reshape(B * H * W, C)
            for kh in range(3) for kw in range(3)]
    lhs = jnp.concatenate(cols, axis=-1)

    y = jnp.dot(lhs, w_ref[...], preferred_element_type=jnp.float32)

    y_ref[...] = y.reshape(B, H, W, Cp).astype(y_ref.dtype)
    stats_ref[...] = jnp.concatenate(
        [jnp.sum(y, axis=0, keepdims=True),
         jnp.sum(y * y, axis=0, keepdims=True)], axis=0)


def _conv3x3(x_nhwc, w_flat, scale=None, shift=None):
    N, H, W, C = x_nhwc.shape
    Cp = w_flat.shape[-1]
    pre = scale is not None
    if not pre:
        scale = jnp.ones((1, C), jnp.float32)
        shift = jnp.zeros((1, C), jnp.float32)

    B = IMG_BLOCK if N % IMG_BLOCK == 0 else 1
    G = N // B
    flops = 2 * N * H * W * (9 * C) * Cp
    bytes_accessed = (x_nhwc.size * x_nhwc.dtype.itemsize + 2 * w_flat.size
                      + 2 * N * H * W * Cp + 4 * G * 2 * Cp)

    y, stats = pl.pallas_call(
        functools.partial(_conv3x3_body, pre_bn_relu=pre),
        out_shape=(jax.ShapeDtypeStruct((N, H, W, Cp), jnp.bfloat16),
                   jax.ShapeDtypeStruct((G, 2, Cp), jnp.float32)),
        grid=(G,),
        in_specs=[
            pl.BlockSpec((B, H, W, C), lambda g: (g, 0, 0, 0)),
            pl.BlockSpec((9 * C, Cp), lambda g: (0, 0)),
            pl.BlockSpec((1, C), lambda g: (0, 0)),
            pl.BlockSpec((1, C), lambda g: (0, 0)),
        ],
        out_specs=(
            pl.BlockSpec((B, H, W, Cp), lambda g: (g, 0, 0, 0)),
            pl.BlockSpec((None, 2, Cp), lambda g: (g, 0, 0)),
        ),
        compiler_params=pltpu.CompilerParams(
            dimension_semantics=("parallel",),
            vmem_limit_bytes=VMEM_LIMIT),
        cost_estimate=pl.CostEstimate(flops=flops, transcendentals=0,
                                      bytes_accessed=bytes_accessed),
    )(x_nhwc, w_flat, scale, shift)
    return y, jnp.sum(stats, axis=0)


def _bn_add_relu_body(y_ref, res_ref, scale_ref, shift_ref, o_ref):
    o_ref[...] = jnp.maximum(
        y_ref[...].astype(jnp.float32) * scale_ref[...] + shift_ref[...]
        + res_ref[...], 0.0)


def _bn_add_relu(y2d, residual, scale, shift):
    M, Cp = y2d.shape
    C = residual.shape[-1]
    tm = M if M <= TILE_M else TILE_M
    return pl.pallas_call(
        _bn_add_relu_body,
        out_shape=jax.ShapeDtypeStruct((M, C), jnp.float32),
        grid=(pl.cdiv(M, tm),),
        in_specs=[pl.BlockSpec((tm, Cp), lambda i: (i, 0)),
                  pl.BlockSpec((tm, C), lambda i: (i, 0)),
                  pl.BlockSpec((1, C), lambda i: (0, 0)),
                  pl.BlockSpec((1, C), lambda i: (0, 0))],
        out_specs=pl.BlockSpec((tm, C), lambda i: (i, 0)),
        compiler_params=pltpu.CompilerParams(
            dimension_semantics=("parallel",),
            vmem_limit_bytes=VMEM_LIMIT),
    )(y2d, residual, scale, shift)


def _pack_w(w_oihw, cin_pad, cout_pad):
    cout, cin = w_oihw.shape[0], w_oihw.shape[1]
    w = jnp.transpose(w_oihw, (2, 3, 1, 0))
    w = jnp.pad(w, ((0, 0), (0, 0), (0, cin_pad - cin), (0, cout_pad - cout)))
    return w.reshape(9 * cin_pad, cout_pad).astype(jnp.bfloat16)


def _bn_scale_shift(stats, gamma, beta, M, out_width):
    C = gamma.shape[0]
    mean = stats[0, :C] / M
    var = jnp.maximum(stats[1, :C] / M - mean * mean, 0.0)
    scale = gamma * jax.lax.rsqrt(var + EPS)
    shift = beta - mean * scale
    if out_width > C:
        scale = jnp.pad(scale, (0, out_width - C))
        shift = jnp.pad(shift, (0, out_width - C))
    return scale.reshape(1, -1), shift.reshape(1, -1)


def kernel(x, w1, g1, b1, w2, g2, b2):
    N, Cin, H, W = x.shape
    Cout = w1.shape[0]
    Cp = ((Cout + 127) // 128) * 128
    M = N * H * W

    xh = jnp.transpose(x, (0, 2, 3, 1))

    y1, stats1 = _conv3x3(xh, _pack_w(w1, Cin, Cp))
    scale1, shift1 = _bn_scale_shift(stats1, g1, b1, M, Cp)

    y2, stats2 = _conv3x3(y1, _pack_w(w2, Cp, Cp),
                          scale=scale1, shift=shift1)
    scale2, shift2 = _bn_scale_shift(stats2, g2, b2, M, Cout)

    out = _bn_add_relu(y2.reshape(M, Cp), xh.reshape(M, Cin),
                       scale2, shift2)
    return jnp.transpose(out.reshape(N, H, W, Cout), (0, 3, 1, 2))
